# baseline (speedup 1.0000x reference)
import functools

import jax
import jax.numpy as jnp
from jax import lax
from jax.experimental import pallas as pl
from jax.experimental.pallas import tpu as pltpu
from jax.experimental.pallas import tpu_sc as plsc

N_LAYERS = 4
K = 1024
DIM = 64
HALF = 256
NSUB = 8
TILE = NSUB * HALF
NCHUNK = 8

_NC = 2
_NS = 16
_NW = _NC * _NS


def _tc_layer(resid, cb, cb_norm):
    x_norm = jnp.sum(resid * resid, axis=1, keepdims=True)
    mm2 = jax.lax.dot_general(
        -2.0 * resid, cb, (((1,), (1,)), ((), ())),
        preferred_element_type=jnp.float32,
    )
    d = (x_norm + cb_norm) + mm2
    d_min = jnp.min(d, axis=1, keepdims=True)
    iota = jax.lax.broadcasted_iota(jnp.int32, d.shape, 1)
    return jnp.min(jnp.where(d == d_min, iota, K), axis=1, keepdims=True)


def _tc_body(x_ref, cb_ref, cbn_ref, out_ref):
    cb, cbn = cb_ref[...], cbn_ref[...]
    for s in range(NSUB):
        code = _tc_layer(x_ref[s * HALF:(s + 1) * HALF], cb, cbn)
        out_ref[s * HALF:(s + 1) * HALF, :] = code


def _tc_codes(resid, cb, cb_norm):
    n = resid.shape[0]
    full = lambda s: pl.BlockSpec(s, lambda i: (0,) * len(s))
    return pl.pallas_call(
        _tc_body,
        grid=(n // TILE,),
        in_specs=[
            pl.BlockSpec((TILE, DIM), lambda i: (i, 0)),
            full((K, DIM)),
            full((1, K)),
        ],
        out_specs=pl.BlockSpec((TILE, 1), lambda i: (i, 0)),
        out_shape=jax.ShapeDtypeStruct((n, 1), jnp.int32),
    )(resid, cb, cb_norm)


def _sc_update(resid, cb_pad, codes):
    n = resid.shape[0]
    b_per_w = n // _NW
    mesh = plsc.VectorSubcoreMesh(core_axis_name="c", subcore_axis_name="s")

    @functools.partial(
        pl.kernel, mesh=mesh,
        out_type=jax.ShapeDtypeStruct((n, DIM), jnp.float32),
        scratch_types=[
            pltpu.VMEM((b_per_w,), jnp.int32),
            pltpu.VMEM((b_per_w, 2 * DIM), jnp.float32),
            pltpu.VMEM((b_per_w, DIM), jnp.float32),
            pltpu.SemaphoreType.DMA,
        ],
    )
    def body(resid_hbm, cb_hbm, idx_hbm, out_hbm, idx_v, rows_v, res_v, sem):
        wid = lax.axis_index("s") * _NC + lax.axis_index("c")
        base = wid * b_per_w
        pltpu.sync_copy(idx_hbm.at[pl.ds(base, b_per_w)], idx_v)
        gather = pltpu.async_copy(cb_hbm.at[idx_v], rows_v, sem)
        pltpu.sync_copy(resid_hbm.at[pl.ds(base, b_per_w)], res_v)
        gather.wait()

        def row(j, carry):
            for k in range(DIM // 16):
                sl = pl.ds(k * 16, 16)
                res_v[j, sl] = res_v[j, sl] - rows_v[j, sl]
            return carry

        lax.fori_loop(0, b_per_w, row, 0)
        pltpu.sync_copy(res_v, out_hbm.at[pl.ds(base, b_per_w)])

    return body(resid, cb_pad, codes)


@jax.jit
def kernel(x, centroids):
    n = x.shape[0]
    cb_norm = jnp.sum(centroids * centroids, axis=2)
    cb_padded = jnp.pad(centroids, ((0, 0), (0, 0), (0, DIM)))
    ch = n // NCHUNK
    resids = [x[c * ch:(c + 1) * ch] for c in range(NCHUNK)]
    codes = [[] for _ in range(NCHUNK)]
    for l in range(N_LAYERS):
        cb = centroids[l]
        cbn = cb_norm[l][None, :]
        for c in range(NCHUNK):
            code = _tc_codes(resids[c], cb, cbn)
            codes[c].append(code)
            if l + 1 < N_LAYERS:
                resids[c] = _sc_update(resids[c], cb_padded[l], code[:, 0])
    return jnp.concatenate(
        [jnp.concatenate(codes[c], axis=1) for c in range(NCHUNK)], axis=0)
